# baseline (speedup 1.0000x reference)
import functools

import jax
import jax.numpy as jnp
from jax import lax
from jax.experimental import pallas as pl
from jax.experimental.pallas import tpu as pltpu
from jax.experimental.pallas import tpu_sc as plsc

_N, _D, _B, _K = 100000, 32, 1024, 32
_M = _B * _K
_NW = 32
_AW = _B // _NW
_PW = _AW * _K


_PITCH = 65


def _sc_fused(embs, idx_all, idx_a, warr):
    mesh = plsc.VectorSubcoreMesh(core_axis_name="c", subcore_axis_name="s")
    vec_t = jax.ShapeDtypeStruct((_M,), jnp.float32)
    grp = 2
    grows = grp * _K
    ngrp = _AW // grp

    tsz = _D * _PITCH + grows

    @functools.partial(
        pl.kernel,
        out_type=tuple(vec_t for _ in range(7)),
        mesh=mesh,
        scratch_types=[
            pltpu.VMEM((128,), jnp.float32),
            pltpu.VMEM((_AW,), jnp.int32),
            pltpu.VMEM((_AW, _D), jnp.float32),
            pltpu.VMEM((_AW, 128), jnp.int32),
            pltpu.VMEM((_AW, 128), jnp.int32),
            pltpu.VMEM((_AW, 128), jnp.int32),
            pltpu.VMEM((_AW, 128), jnp.int32),
            pltpu.VMEM((4, 4, grows, _D), jnp.float32),
            pltpu.VMEM((4, tsz), jnp.float32),
            pltpu.VMEM((7, _PW), jnp.float32),
            pltpu.SemaphoreType.DMA,
            pltpu.SemaphoreType.DMA,
            pltpu.SemaphoreType.DMA,
            pltpu.SemaphoreType.DMA,
        ],
        compiler_params=pltpu.CompilerParams(
            use_tc_tiling_on_sc=False, needs_layout_passes=False),
    )
    def k(table, iall, ia, wh,
          o_lp, o_ln, o_rsp, o_rsn, o_s2p, o_s2n, o_s1,
          wv, av, zbuf, ixp, ixn, ixsp, ixsn, bufs, tbuf, obuf,
          sem0, sem1, sem2, sem3):
        wid = lax.axis_index("s") * 2 + lax.axis_index("c")
        pltpu.sync_copy(wh, wv)
        abase = wid * _AW
        pltpu.sync_copy(ia.at[pl.ds(abase, _AW)], av)
        pltpu.async_copy(table.at[av], zbuf, sem0).wait()
        rbase = wid * _AW
        pltpu.sync_copy(iall.at[pl.ds(0 * _B + rbase, _AW), :], ixp)
        pltpu.sync_copy(iall.at[pl.ds(1 * _B + rbase, _AW), :], ixn)
        pltpu.sync_copy(iall.at[pl.ds(2 * _B + rbase, _AW), :], ixsp)
        pltpu.sync_copy(iall.at[pl.ds(3 * _B + rbase, _AW), :], ixsn)
        ixs = (ixp, ixn, ixsp, ixsn)
        base = wid * _PW

        fc1 = (wv[pl.ds(0, 16)], wv[pl.ds(16, 16)])
        fc2 = (wv[pl.ds(32, 16)], wv[pl.ds(48, 16)])
        s1w = (wv[pl.ds(64, 16)], wv[pl.ds(80, 16)])
        s2w = (wv[pl.ds(96, 16)], wv[pl.ds(112, 16)])
        r16 = lax.iota(jnp.int32, 16)
        halves = (r16, r16 + 16)
        i_pitch = r16 * _PITCH
        zero = jnp.zeros((16,), jnp.float32)
        sems = (sem0, sem1, sem2, sem3)

        def fire(g, slot):
            for t in range(4):
                for j in range(grp):
                    pltpu.async_copy(
                        table.at[ixs[t].at[g * grp + j, pl.ds(0, _K)]],
                        bufs.at[slot, t].at[pl.ds(j * _K, _K)], sems[slot])

        def drain(g, slot):
            for t in range(4):
                for j in range(grp):
                    pltpu.make_async_copy(
                        table.at[ixs[t].at[g * grp + j, pl.ds(0, _K)]],
                        bufs.at[slot, t].at[pl.ds(j * _K, _K)],
                        sems[slot]).wait()

        def transpose(slot):
            @plsc.parallel_loop(0, grows, step=1, unroll=4)
            def _(r):
                rf = jnp.full((16,), r, jnp.int32)
                for h in range(2):
                    dst = i_pitch + (jnp.full((16,), h * 16 * _PITCH,
                                              jnp.int32) + rf)
                    for t in range(4):
                        v = plsc.load_gather(bufs.at[slot, t],
                                             [rf, halves[h]])
                        plsc.store_scatter(tbuf.at[t], [dst], v)

        gdn = lax.GatherDimensionNumbers(
            offset_dims=(), collapsed_slice_dims=(0,), start_index_map=(0,))

        def take16(vpair, d):
            src = vpair[d // 16]
            idx = jnp.full((16, 1), d % 16, jnp.int32)
            return lax.gather(src, idx, gdn, slice_sizes=(1,),
                              mode=lax.GatherScatterMode.PROMISE_IN_BOUNDS)

        def compute(g, slot):
            transpose(slot)
            for a_local in range(grp):
                a = g * grp + a_local
                af = jnp.full((16,), a, jnp.int32)
                z0 = plsc.load_gather(zbuf, [af, halves[0]])
                z1v = plsc.load_gather(zbuf, [af, halves[1]])
                rs_z = jnp.sum(z0 * fc1[0] + z1v * fc1[1])
                s1_d = jnp.sum(z0 * s1w[0] + z1v * s1w[1])
                zpair = (z0, z1v)
                acc = [[zero] * 2 for _ in range(6)]
                for d in range(_D):
                    zc = take16(zpair, d)
                    fc2d = take16(fc2, d)
                    s2wd = take16(s2w, d)
                    for h in range(2):
                        off = d * _PITCH + a_local * _K + h * 16
                        cp_ = tbuf[0, pl.ds(off, 16)]
                        acc[0][h] = acc[0][h] + cp_ * zc
                        cn_ = tbuf[1, pl.ds(off, 16)]
                        acc[1][h] = acc[1][h] + cn_ * zc
                        sp_ = tbuf[2, pl.ds(off, 16)]
                        acc[2][h] = acc[2][h] + sp_ * fc2d
                        acc[4][h] = acc[4][h] + sp_ * s2wd
                        sn_ = tbuf[3, pl.ds(off, 16)]
                        acc[3][h] = acc[3][h] + sn_ * fc2d
                        acc[5][h] = acc[5][h] + sn_ * s2wd
                s1_v = jnp.full((16,), s1_d)
                for h in range(2):
                    off = a * _K + 16 * h
                    obuf[0, pl.ds(off, 16)] = acc[0][h]
                    obuf[1, pl.ds(off, 16)] = acc[1][h]
                    obuf[2, pl.ds(off, 16)] = acc[2][h] + rs_z
                    obuf[3, pl.ds(off, 16)] = acc[3][h] + rs_z
                    obuf[4, pl.ds(off, 16)] = acc[4][h]
                    obuf[5, pl.ds(off, 16)] = acc[5][h]
                    obuf[6, pl.ds(off, 16)] = s1_v

        fire(0, 0)
        fire(1, 1)
        fire(2, 2)

        def quad_step(q, _):
            for u in range(4):
                g = 4 * q + u
                drain(g, u)

                @pl.when(g + 3 < ngrp)
                def _(g=g, u=u):
                    fire(g + 3, (u + 3) % 4)

                compute(g, u)
            return 0

        lax.fori_loop(0, ngrp // 4, quad_step, 0)
        outs = (o_lp, o_ln, o_rsp, o_rsn, o_s2p, o_s2n, o_s1)
        for i, oref in enumerate(outs):
            pltpu.sync_copy(obuf.at[i], oref.at[pl.ds(base, _PW)])

    return k(embs, idx_all, idx_a, warr)


def _tc_prep_idx(ip, ng, isp, isn):
    def body(ip_ref, ng_ref, isp_ref, isn_ref, out_ref):
        pad = jnp.zeros((_B, 128 - _K), jnp.int32)
        for t, r in enumerate((ip_ref, ng_ref, isp_ref, isn_ref)):
            out_ref[pl.ds(t * _B, _B), :] = jnp.concatenate(
                [r[...], pad], axis=1)

    return pl.pallas_call(
        body,
        out_shape=jax.ShapeDtypeStruct((4 * _B, 128), jnp.int32),
    )(ip, ng, isp, isn)


def _tc_loss(lp, ln, rsp, rsn, s2p, s2n, s1d, pw, nw, bvec):
    def body(bv_ref, lp_ref, ln_ref, rsp_ref, rsn_ref, s2p_ref, s2n_ref,
             s1_ref, pw_ref, nw_ref, out_ref):
        fcb, s1b, s2b = bv_ref[0], bv_ref[1], bv_ref[2]
        sp = jax.nn.softplus
        sg = jax.nn.sigmoid
        invk = 1.0 / _K
        s1 = sg(s1_ref[...] + s1b)
        dp = s1 - sg(s2p_ref[...] + s2b)
        tp = jnp.minimum(dp, -0.5) - dp
        dn = s1 - sg(s2n_ref[...] + s2b)
        tn = jnp.maximum(dn, 0.5) - dn
        term = (sp(-lp_ref[...]) + pw_ref[...] * sp(-(rsp_ref[...] + fcb))
                + sp(ln_ref[...]) + nw_ref[...] * sp(rsn_ref[...] + fcb))
        total = invk * jnp.sum(term) + 5.0 * jnp.sum(tp * tp + tn * tn)
        out_ref[0, 0] = total

    vspec = pl.BlockSpec((_M // 128, 128), lambda: (0, 0))
    return pl.pallas_call(
        body,
        in_specs=[pl.BlockSpec(memory_space=pltpu.SMEM)] + [vspec] * 9,
        out_specs=pl.BlockSpec(memory_space=pltpu.SMEM),
        out_shape=jax.ShapeDtypeStruct((1, 1), jnp.float32),
    )(bvec, lp, ln, rsp, rsn, s2p, s2n, s1d, pw, nw)


def kernel(embs, fc_w, fc_b, s1_w, s1_b, s2_w, s2_b, pos_w, neg_w,
           anchors, pos_idx, neg_idx, sta_pos_idx, sta_neg_idx):
    i32 = jnp.int32
    warr = jnp.concatenate([fc_w[0, :_D], fc_w[0, _D:], s1_w[0], s2_w[0]])
    idx_all = _tc_prep_idx(
        pos_idx.astype(i32), neg_idx.astype(i32), sta_pos_idx.astype(i32),
        sta_neg_idx.astype(i32))
    outs = _sc_fused(embs, idx_all, anchors.astype(i32), warr)
    shaped = [o.reshape(_M // 128, 128) for o in outs]
    bvec = jnp.stack([fc_b[0], s1_b[0], s2_b[0]])
    out = _tc_loss(*shaped,
                   pos_w.reshape(_M // 128, 128),
                   neg_w.reshape(_M // 128, 128), bvec)
    return out[0, 0]


# speedup vs baseline: 1.0807x; 1.0807x over previous
import functools

import jax
import jax.numpy as jnp
from jax import lax
from jax.experimental import pallas as pl
from jax.experimental.pallas import tpu as pltpu
from jax.experimental.pallas import tpu_sc as plsc

_N, _D, _B, _K = 100000, 32, 1024, 32
_M = _B * _K
_NW = 32
_AW = _B // _NW
_PW = _AW * _K
_PITCH = 65


def _sc_fused(embs, idx_all, warr):
    mesh = plsc.VectorSubcoreMesh(core_axis_name="c", subcore_axis_name="s")
    vec_t = jax.ShapeDtypeStruct((_M,), jnp.float32)
    grp = 2
    grows = grp * _K
    ngrp = _AW // grp
    tsz = _D * _PITCH + grows

    @functools.partial(
        pl.kernel,
        out_type=tuple(vec_t for _ in range(7)),
        mesh=mesh,
        scratch_types=[
            pltpu.VMEM((128,), jnp.float32),
            pltpu.VMEM((_AW,), jnp.int32),
            pltpu.VMEM((_AW, _D), jnp.float32),
            pltpu.VMEM((_PW,), jnp.int32),
            pltpu.VMEM((_PW,), jnp.int32),
            pltpu.VMEM((_PW,), jnp.int32),
            pltpu.VMEM((_PW,), jnp.int32),
            pltpu.VMEM((2, 4, grows, _D), jnp.float32),
            pltpu.VMEM((4, tsz), jnp.float32),
            pltpu.VMEM((7, _PW), jnp.float32),
            pltpu.SemaphoreType.DMA,
            pltpu.SemaphoreType.DMA,
        ],
        compiler_params=pltpu.CompilerParams(
            use_tc_tiling_on_sc=False, needs_layout_passes=False),
    )
    def k(table, iall, wh,
          o_lp, o_ln, o_rsp, o_rsn, o_s2p, o_s2n, o_s1,
          wv, av, zbuf, ixp, ixn, ixsp, ixsn, bufs, tbuf, obuf, sem0, sem1):
        wid = lax.axis_index("s") * 2 + lax.axis_index("c")
        pltpu.sync_copy(wh, wv)
        abase = wid * _AW
        pltpu.sync_copy(iall.at[pl.ds(4 * _M + abase, _AW)], av)
        pltpu.async_copy(table.at[av], zbuf, sem0).wait()
        base = wid * _PW
        pltpu.sync_copy(iall.at[pl.ds(0 * _M + base, _PW)], ixp)
        pltpu.sync_copy(iall.at[pl.ds(1 * _M + base, _PW)], ixn)
        pltpu.sync_copy(iall.at[pl.ds(2 * _M + base, _PW)], ixsp)
        pltpu.sync_copy(iall.at[pl.ds(3 * _M + base, _PW)], ixsn)
        ixs = (ixp, ixn, ixsp, ixsn)

        fc1 = (wv[pl.ds(0, 16)], wv[pl.ds(16, 16)])
        fc2 = (wv[pl.ds(32, 16)], wv[pl.ds(48, 16)])
        s1w = (wv[pl.ds(64, 16)], wv[pl.ds(80, 16)])
        s2w = (wv[pl.ds(96, 16)], wv[pl.ds(112, 16)])
        r16 = lax.iota(jnp.int32, 16)
        halves = (r16, r16 + 16)
        i_pitch = r16 * _PITCH
        zero = jnp.zeros((16,), jnp.float32)
        sems = (sem0, sem1)

        def fire(g, slot):
            o = g * grows
            for t in range(4):
                pltpu.async_copy(
                    table.at[ixs[t].at[pl.ds(o, grows)]],
                    bufs.at[slot, t], sems[slot])

        def drain(g, slot):
            o = g * grows
            for t in range(4):
                pltpu.make_async_copy(
                    table.at[ixs[t].at[pl.ds(o, grows)]],
                    bufs.at[slot, t], sems[slot]).wait()

        def transpose(slot):
            @plsc.parallel_loop(0, grows, step=1, unroll=4)
            def _(r):
                rf = jnp.full((16,), r, jnp.int32)
                for h in range(2):
                    dst = i_pitch + (jnp.full((16,), h * 16 * _PITCH,
                                              jnp.int32) + rf)
                    for t in range(4):
                        v = plsc.load_gather(bufs.at[slot, t],
                                             [rf, halves[h]])
                        plsc.store_scatter(tbuf.at[t], [dst], v)

        gdn = lax.GatherDimensionNumbers(
            offset_dims=(), collapsed_slice_dims=(0,), start_index_map=(0,))

        def take16(vpair, d):
            src = vpair[d // 16]
            idx = jnp.full((16, 1), d % 16, jnp.int32)
            return lax.gather(src, idx, gdn, slice_sizes=(1,),
                              mode=lax.GatherScatterMode.PROMISE_IN_BOUNDS)

        def compute(g, slot):
            transpose(slot)
            for a_local in range(grp):
                a = g * grp + a_local
                af = jnp.full((16,), a, jnp.int32)
                z0 = plsc.load_gather(zbuf, [af, halves[0]])
                z1v = plsc.load_gather(zbuf, [af, halves[1]])
                rs_z = jnp.sum(z0 * fc1[0] + z1v * fc1[1])
                s1_d = jnp.sum(z0 * s1w[0] + z1v * s1w[1])
                zpair = (z0, z1v)
                acc = [[zero] * 2 for _ in range(6)]
                for d in range(_D):
                    zc = take16(zpair, d)
                    fc2d = take16(fc2, d)
                    s2wd = take16(s2w, d)
                    for h in range(2):
                        off = d * _PITCH + a_local * _K + h * 16
                        cp_ = tbuf[0, pl.ds(off, 16)]
                        acc[0][h] = acc[0][h] + cp_ * zc
                        cn_ = tbuf[1, pl.ds(off, 16)]
                        acc[1][h] = acc[1][h] + cn_ * zc
                        sp_ = tbuf[2, pl.ds(off, 16)]
                        acc[2][h] = acc[2][h] + sp_ * fc2d
                        acc[4][h] = acc[4][h] + sp_ * s2wd
                        sn_ = tbuf[3, pl.ds(off, 16)]
                        acc[3][h] = acc[3][h] + sn_ * fc2d
                        acc[5][h] = acc[5][h] + sn_ * s2wd
                s1_v = jnp.full((16,), s1_d)
                for h in range(2):
                    off = a * _K + 16 * h
                    obuf[0, pl.ds(off, 16)] = acc[0][h]
                    obuf[1, pl.ds(off, 16)] = acc[1][h]
                    obuf[2, pl.ds(off, 16)] = acc[2][h] + rs_z
                    obuf[3, pl.ds(off, 16)] = acc[3][h] + rs_z
                    obuf[4, pl.ds(off, 16)] = acc[4][h]
                    obuf[5, pl.ds(off, 16)] = acc[5][h]
                    obuf[6, pl.ds(off, 16)] = s1_v

        fire(0, 0)

        def pair_step(p, _):
            g0 = 2 * p
            fire(g0 + 1, 1)
            drain(g0, 0)
            compute(g0, 0)

            @pl.when(g0 + 2 < ngrp)
            def _():
                fire(g0 + 2, 0)

            drain(g0 + 1, 1)
            compute(g0 + 1, 1)
            return 0

        lax.fori_loop(0, ngrp // 2, pair_step, 0)
        outs = (o_lp, o_ln, o_rsp, o_rsn, o_s2p, o_s2n, o_s1)
        for i, oref in enumerate(outs):
            pltpu.sync_copy(obuf.at[i], oref.at[pl.ds(base, _PW)])

    return k(embs, idx_all, warr)


def _tc_loss(lp, ln, rsp, rsn, s2p, s2n, s1d, pw, nw, bvec):
    def body(bv_ref, lp_ref, ln_ref, rsp_ref, rsn_ref, s2p_ref, s2n_ref,
             s1_ref, pw_ref, nw_ref, out_ref):
        fcb, s1b, s2b = bv_ref[0], bv_ref[1], bv_ref[2]
        sp = jax.nn.softplus
        sg = jax.nn.sigmoid
        invk = 1.0 / _K
        s1 = sg(s1_ref[...] + s1b)
        dp = s1 - sg(s2p_ref[...] + s2b)
        tp = jnp.minimum(dp, -0.5) - dp
        dn = s1 - sg(s2n_ref[...] + s2b)
        tn = jnp.maximum(dn, 0.5) - dn
        term = (sp(-lp_ref[...]) + pw_ref[...] * sp(-(rsp_ref[...] + fcb))
                + sp(ln_ref[...]) + nw_ref[...] * sp(rsn_ref[...] + fcb))
        total = invk * jnp.sum(term) + 5.0 * jnp.sum(tp * tp + tn * tn)
        out_ref[0, 0] = total

    vspec = pl.BlockSpec((_M // 128, 128), lambda: (0, 0))
    return pl.pallas_call(
        body,
        in_specs=[pl.BlockSpec(memory_space=pltpu.SMEM)] + [vspec] * 9,
        out_specs=pl.BlockSpec(memory_space=pltpu.SMEM),
        out_shape=jax.ShapeDtypeStruct((1, 1), jnp.float32),
    )(bvec, lp, ln, rsp, rsn, s2p, s2n, s1d, pw, nw)


def kernel(embs, fc_w, fc_b, s1_w, s1_b, s2_w, s2_b, pos_w, neg_w,
           anchors, pos_idx, neg_idx, sta_pos_idx, sta_neg_idx):
    i32 = jnp.int32
    warr = jnp.concatenate([fc_w[0, :_D], fc_w[0, _D:], s1_w[0], s2_w[0]])
    idx_all = jnp.concatenate([
        pos_idx.reshape(-1), neg_idx.reshape(-1),
        sta_pos_idx.reshape(-1), sta_neg_idx.reshape(-1),
        anchors]).astype(i32)
    outs = _sc_fused(embs, idx_all, warr)
    shaped = [o.reshape(_M // 128, 128) for o in outs]
    bvec = jnp.stack([fc_b[0], s1_b[0], s2_b[0]])
    out = _tc_loss(*shaped,
                   pos_w.reshape(_M // 128, 128),
                   neg_w.reshape(_M // 128, 128), bvec)
    return out[0, 0]
